# Initial kernel scaffold; baseline (speedup 1.0000x reference)
import jax
import jax.numpy as jnp
from jax.experimental import pallas as pl


def kernel(x, Win, bin_, g_in, b_in, Wr, br, W1, b1, W2, b2, g_moe, b_moe, g_out, b_out, Wc, bc):
    raise NotImplementedError("write your pallas kernel here")


# speedup vs baseline: 1.3499x; 1.3499x over previous
import functools

import jax
import jax.numpy as jnp
from jax.experimental import pallas as pl
from jax.experimental.pallas import tpu as pltpu

_HIGH = jax.lax.Precision.HIGHEST


def _layernorm(x, g, b, eps=1e-5):
    m = jnp.mean(x, axis=-1, keepdims=True)
    v = jnp.mean((x - m) ** 2, axis=-1, keepdims=True)
    return (x - m) / jnp.sqrt(v + eps) * g + b


def _router_kernel(x_ref, Win_ref, bin_ref, gin_ref, bim_ref, Wr_ref, br_ref,
                   h_ref, gates_ref):
    x = x_ref[...]
    h = jax.lax.dot_general(x.astype(jnp.bfloat16),
                            Win_ref[...].astype(jnp.bfloat16),
                            (((1,), (0,)), ((), ())),
                            preferred_element_type=jnp.float32)
    h = h + bin_ref[...][None, :]
    h = _layernorm(h, gin_ref[...][None, :], bim_ref[...][None, :])
    h_ref[...] = h
    logits = jax.lax.dot_general(h.astype(jnp.bfloat16),
                                 Wr_ref[...].astype(jnp.bfloat16),
                                 (((1,), (0,)), ((), ())),
                                 preferred_element_type=jnp.float32)
    logits = logits + br_ref[...][None, :]
    E = logits.shape[-1]
    ei = jax.lax.broadcasted_iota(jnp.int32, logits.shape, 1)
    v1 = jnp.max(logits, axis=-1, keepdims=True)
    i1 = jnp.min(jnp.where(logits == v1, ei, E), axis=-1, keepdims=True)
    l2 = jnp.where(ei == i1, -jnp.inf, logits)
    v2 = jnp.max(l2, axis=-1, keepdims=True)
    i2 = jnp.min(jnp.where(l2 == v2, ei, E), axis=-1, keepdims=True)
    p1 = 1.0 / (1.0 + jnp.exp(v2 - v1))
    p2 = 1.0 - p1
    gates_ref[...] = jnp.where(ei == i1, p1, 0.0) + jnp.where(ei == i2, p2, 0.0)


def _moe_dense_kernel(h_ref, W1_ref, b1_ref, W2_ref, b2_ref, gates_ref,
                      acc_ref):
    e = pl.program_id(1)

    @pl.when(e == 0)
    def _init():
        acc_ref[...] = jnp.zeros_like(acc_ref)

    hb = h_ref[...].astype(jnp.bfloat16)
    u = jax.lax.dot_general(hb, W1_ref[0], (((1,), (0,)), ((), ())),
                            preferred_element_type=jnp.float32)
    u = jax.nn.gelu(u + b1_ref[0, 0][None, :])
    o = jax.lax.dot_general(u.astype(jnp.bfloat16), W2_ref[0],
                            (((1,), (0,)), ((), ())),
                            preferred_element_type=jnp.float32)
    o = o + b2_ref[0, 0][None, :]
    gates = gates_ref[...]
    ei = jax.lax.broadcasted_iota(jnp.int32, gates.shape, 1)
    gate_e = jnp.sum(jnp.where(ei == e, gates, 0.0), axis=-1, keepdims=True)
    acc_ref[...] += gate_e * o


def _final_kernel(h_ref, acc_ref, gmo_ref, bmo_ref, gou_ref, bou_ref,
                  Wc_ref, bc_ref, out_ref):
    moe = _layernorm(h_ref[...] + acc_ref[...], gmo_ref[...][None, :],
                     bmo_ref[...][None, :])
    final = _layernorm(moe, gou_ref[...][None, :], bou_ref[...][None, :])
    out = jax.lax.dot_general(final.astype(jnp.bfloat16), Wc_ref[...],
                              (((1,), (0,)), ((), ())),
                              preferred_element_type=jnp.float32)
    out_ref[...] = out + bc_ref[...][None, :]


def kernel(x, Win, bin_, g_in, b_in, Wr, br, W1, b1, W2, b2, g_moe, b_moe,
           g_out, b_out, Wc, bc):
    N, D = x.shape
    E, _, H = W1.shape
    C = Wc.shape[1]

    h, gates = pl.pallas_call(
        _router_kernel,
        out_shape=(jax.ShapeDtypeStruct((N, D), jnp.float32),
                   jax.ShapeDtypeStruct((N, E), jnp.float32)),
    )(x, Win, bin_, g_in, b_in, Wr, br)

    TB = 512
    nt = N // TB
    W1b = W1.astype(jnp.bfloat16)
    W2b = W2.astype(jnp.bfloat16)
    b1r = b1.reshape(E, 1, H)
    b2r = b2.reshape(E, 1, D)
    acc = pl.pallas_call(
        _moe_dense_kernel,
        grid=(nt, E),
        in_specs=[
            pl.BlockSpec((TB, D), lambda t, e: (t, 0)),
            pl.BlockSpec((1, D, H), lambda t, e: (e, 0, 0)),
            pl.BlockSpec((1, 1, H), lambda t, e: (e, 0, 0)),
            pl.BlockSpec((1, H, D), lambda t, e: (e, 0, 0)),
            pl.BlockSpec((1, 1, D), lambda t, e: (e, 0, 0)),
            pl.BlockSpec((TB, E), lambda t, e: (t, 0)),
        ],
        out_specs=pl.BlockSpec((TB, D), lambda t, e: (t, 0)),
        out_shape=jax.ShapeDtypeStruct((N, D), jnp.float32),
    )(h, W1b, b1r, W2b, b2r, gates)

    Wcb = Wc.astype(jnp.bfloat16)
    out = pl.pallas_call(
        _final_kernel,
        grid=(nt,),
        in_specs=[
            pl.BlockSpec((TB, D), lambda t: (t, 0)),
            pl.BlockSpec((TB, D), lambda t: (t, 0)),
            pl.BlockSpec((D,), lambda t: (0,)),
            pl.BlockSpec((D,), lambda t: (0,)),
            pl.BlockSpec((D,), lambda t: (0,)),
            pl.BlockSpec((D,), lambda t: (0,)),
            pl.BlockSpec((D, C), lambda t: (0, 0)),
            pl.BlockSpec((C,), lambda t: (0,)),
        ],
        out_specs=pl.BlockSpec((TB, C), lambda t: (t, 0)),
        out_shape=jax.ShapeDtypeStruct((N, C), jnp.float32),
    )(h, acc, g_moe, b_moe, g_out, b_out, Wcb, bc)
    return out
